# speedup vs baseline: 6.0532x; 6.0532x over previous
import functools

import jax
import jax.numpy as jnp
from jax import lax
from jax.experimental import pallas as pl
from jax.experimental.pallas import tpu as pltpu
from jax.experimental.pallas import tpu_sc as plsc

_EMBED = 128
_SEQLEN = 200
_NSEG = 3
_NC = 2
_NS = 16
_NW = _NC * _NS
_LANES = 16
_CHUNK = 128


def _pe_seg_body(seg_ref, out_ref):
    rows = _SEQLEN * _NSEG
    r = lax.broadcasted_iota(jnp.int32, (rows, _EMBED), 0)
    c = lax.broadcasted_iota(jnp.int32, (rows, _EMBED), 1)
    pos = (r // _NSEG).astype(jnp.float32)
    g = r % _NSEG
    i_even = ((c // 2) * 2).astype(jnp.float32)
    div = jnp.exp(-jnp.log(10000.0) * i_even / _EMBED)
    ang = pos * div
    pe = jnp.where(c % 2 == 0, jnp.sin(ang), jnp.cos(ang))
    seg = jnp.where(
        g == 0,
        seg_ref[0:1, :],
        jnp.where(g == 1, seg_ref[1:2, :], seg_ref[2:3, :]),
    )
    out_ref[...] = pe + seg


def _build_pe_seg(segment_table):
    return pl.pallas_call(
        _pe_seg_body,
        out_shape=jax.ShapeDtypeStruct((_SEQLEN * _NSEG, _EMBED), jnp.float32),
    )(segment_table)


def _make_sc_kernel(n_rows):
    rows_per_w = n_rows // _NW
    n_chunks = rows_per_w // _CHUNK
    mesh = plsc.VectorSubcoreMesh(core_axis_name="c", subcore_axis_name="s")

    @functools.partial(
        pl.kernel,
        out_type=jax.ShapeDtypeStruct((n_rows, _EMBED), jnp.float32),
        mesh=mesh,
        scratch_types=[
            pltpu.VMEM((_CHUNK,), jnp.int32),
            pltpu.VMEM((_CHUNK,), jnp.int32),
            pltpu.VMEM((_CHUNK, _EMBED), jnp.float32),
            pltpu.VMEM((_CHUNK, _EMBED), jnp.float32),
            pltpu.SemaphoreType.DMA,
            pltpu.SemaphoreType.DMA,
        ],
    )
    def sc_embed(seq_hbm, lbl_hbm, tok_hbm, peseg_hbm, out_hbm,
                 idx_tok, idx_ps, rows_tok, rows_ps, sem_a, sem_b):
        wid = lax.axis_index("s") * _NC + lax.axis_index("c")
        wbase = wid * rows_per_w

        @pl.loop(0, n_chunks)
        def _chunk(ci):
            base = wbase + ci * _CHUNK
            pltpu.sync_copy(seq_hbm.at[pl.ds(base, _CHUNK)], idx_tok)
            pltpu.sync_copy(lbl_hbm.at[pl.ds(base, _CHUNK)], idx_ps)

            @pl.loop(0, _CHUNK // _LANES)
            def _fuse(j):
                off = base + j * _LANES
                pos = (off + lax.iota(jnp.int32, _LANES)) % _SEQLEN
                sl = pl.ds(j * _LANES, _LANES)
                idx_ps[sl] = pos * _NSEG + idx_ps[sl]

            cp_a = pltpu.async_copy(tok_hbm.at[idx_tok], rows_tok, sem_a)
            cp_b = pltpu.async_copy(peseg_hbm.at[idx_ps], rows_ps, sem_b)
            cp_a.wait()
            cp_b.wait()

            @pl.loop(0, _CHUNK)
            def _add(i):
                for cg in range(_EMBED // _LANES):
                    sl = pl.ds(cg * _LANES, _LANES)
                    plsc.addupdate(rows_tok.at[i, sl], rows_ps[i, sl])

            pltpu.sync_copy(rows_tok, out_hbm.at[pl.ds(base, _CHUNK)])

    return sc_embed


def kernel(sequence, segment_label, token_table, segment_table):
    batch, seqlen = sequence.shape
    n_rows = batch * seqlen
    pe_seg = _build_pe_seg(segment_table.astype(jnp.float32))
    seq_flat = sequence.reshape(-1).astype(jnp.int32)
    lbl_flat = segment_label.reshape(-1).astype(jnp.int32)
    sc = _make_sc_kernel(n_rows)
    out = sc(seq_flat, lbl_flat, token_table, pe_seg)
    return out.reshape(batch, seqlen, _EMBED)
